# baseline (speedup 1.0000x reference)
import dataclasses
import functools

import jax
import jax.numpy as jnp
from jax import lax
from jax.experimental import pallas as pl
from jax.experimental.pallas import tpu as pltpu
from jax.experimental.pallas import tpu_sc as plsc

NROW = 32
NCOL = 1_000_000
NSAMP = 64
BC = 128
SUB = 8
CB = SUB * BC
NCB = 128
CC = NCB * CB
NSTEP = -(-NCOL // CC)
NG = NROW // 8
GI = NG * NSTEP
NB = -(-NCOL // CB)
NBP = NSTEP * NCB
MAXA = (NCOL // BC) * BC
L = 16


def _pass1_body(x_ref, bs_ref):
    i = pl.program_id(1)

    def emit(mask_cols):
        for b in range(NCB):
            acc = x_ref[:, b * CB:b * CB + BC]
            for j in range(1, SUB):
                sl = x_ref[:, b * CB + j * BC:b * CB + (j + 1) * BC]
                if mask_cols:
                    col = (i * CC + b * CB + j * BC
                           + lax.broadcasted_iota(jnp.int32, (8, BC), 1))
                    sl = jnp.where(col < NCOL, sl, 0.0)
                acc = acc + sl
            if mask_cols:
                col = (i * CC + b * CB
                       + lax.broadcasted_iota(jnp.int32, (8, BC), 1))
                acc = jnp.where(col < NCOL, acc, 0.0)
            bs_ref[0, :, b:b + 1] = jnp.sum(acc, axis=-1, keepdims=True)

    @pl.when(i < NSTEP - 1)
    def _full():
        emit(False)

    @pl.when(i == NSTEP - 1)
    def _masked():
        emit(True)


def _pass1(x):
    return pl.pallas_call(
        _pass1_body,
        grid=(NG, NSTEP),
        in_specs=[pl.BlockSpec((8, CC), lambda g, i: (g, i))],
        out_specs=pl.BlockSpec((1, 8, NCB), lambda g, i: (g * NSTEP + i, 0, 0)),
        out_shape=jax.ShapeDtypeStruct((GI, 8, NCB), jnp.float32),
    )(x)


def _sc_compiler_params():
    cp = pltpu.CompilerParams(use_tc_tiling_on_sc=True)
    if "needs_layout_passes" in pltpu.CompilerParams.__dataclass_fields__:
        cp = dataclasses.replace(cp, needs_layout_passes=False)
    return cp


def _sample_body(xl_hbm, bs_hbm, u_hbm, o_hbm, bs_v, bcum_v, t_v, blk_v,
                 res_v, sem, sem_b):
    r = lax.axis_index("s") * 2 + lax.axis_index("c")
    g = r // 8
    s = r % 8
    iota = lax.iota(jnp.int32, L)

    copies = [
        pltpu.async_copy(bs_hbm.at[g * NSTEP + i, s],
                         bs_v.at[pl.ds(i * NCB, NCB)], sem)
        for i in range(NSTEP)
    ]
    for cp in copies:
        cp.wait()

    def cum_body(i, carry):
        c = plsc.cumsum(bs_v[pl.ds(i * L, L)]) + carry
        bcum_v[pl.ds(i * L, L)] = c
        return jnp.max(c)

    total = lax.fori_loop(0, NBP // L, cum_body, jnp.float32(0.0))

    pltpu.sync_copy(u_hbm.at[r], t_v)
    for j in range(NSAMP // L):
        t_v[pl.ds(j * L, L)] = t_v[pl.ds(j * L, L)] * total

    def search(k):
        kk = jnp.full((L,), k, jnp.int32)
        tb = plsc.load_gather(t_v, [kk])
        lo = jnp.int32(0)
        basev = jnp.zeros((L,), jnp.float32)
        for st in (64, 4, 1):
            p = jnp.minimum(lo + (iota + 1) * st - 1, NBP - 1)
            v = plsc.load_gather(bcum_v, [p])
            le = v <= tb
            cnt = jnp.sum(le.astype(jnp.int32))
            basev = jnp.maximum(basev, jnp.where(le, v, 0.0))
            lo = lo + cnt * st
        return jnp.minimum(lo, NB - 1), jnp.max(basev), jnp.max(tb)

    def issue(par, block, dsem):
        a = block * CB
        return [
            pltpu.async_copy(
                xl_hbm.at[r, pl.ds(jnp.minimum(a + j * BC, MAXA), BC)],
                blk_v.at[pl.ds(par + j * BC, BC)], dsem)
            for j in range(SUB)
        ]

    def scan(k, par, block, base, t_s):
        tb = jnp.full((L,), t_s)
        a = block * CB
        pre = base
        nfull = jnp.int32(0)
        fbase = base
        for j in range(SUB):
            acc = blk_v[pl.ds(par + j * BC, L)]
            for gg in range(1, BC // L):
                acc = acc + blk_v[pl.ds(par + j * BC + gg * L, L)]
            ok = (a + j * BC) < NCOL
            pre2 = pre + jnp.where(ok, jnp.sum(acc), 0.0)
            lt = pre2 <= t_s
            nfull = nfull + jnp.where(lt, 1, 0)
            fbase = jnp.where(lt, pre2, fbase)
            pre = pre2

        off = par + jnp.minimum(nfull, SUB - 1) * BC
        cnt2 = jnp.int32(0)
        carry2 = fbase
        for gg in range(BC // L):
            cs = plsc.cumsum(blk_v[pl.ds(off + gg * L, L)]) + carry2
            cnt2 = cnt2 + jnp.sum((cs <= tb).astype(jnp.int32))
            carry2 = jnp.max(cs)

        final = jnp.minimum(a + nfull * BC + cnt2, NCOL - 1)
        plsc.store_scatter(res_v, [jnp.full((L,), k, jnp.int32)],
                           jnp.full((L,), final, jnp.int32),
                           mask=iota == 0)

    def pair_body(m, acc_):
        k0 = 2 * m
        b0, ba0, t0 = search(k0)
        h0 = issue(0, b0, sem)
        b1, ba1, t1 = search(k0 + 1)
        h1 = issue(CB, b1, sem_b)
        for h in h0:
            h.wait()
        scan(k0, 0, b0, ba0, t0)
        for h in h1:
            h.wait()
        scan(k0 + 1, CB, b1, ba1, t1)
        return acc_

    lax.fori_loop(0, NSAMP // 2, pair_body, jnp.int32(0))
    pltpu.sync_copy(res_v, o_hbm.at[r])


@functools.partial(
    pl.kernel,
    out_type=jax.ShapeDtypeStruct((NROW, NSAMP), jnp.int32),
    mesh=plsc.VectorSubcoreMesh(core_axis_name="c", subcore_axis_name="s"),
    scratch_types=[
        pltpu.VMEM((NBP,), jnp.float32),
        pltpu.VMEM((NBP,), jnp.float32),
        pltpu.VMEM((NSAMP,), jnp.float32),
        pltpu.VMEM((2 * SUB * BC,), jnp.float32),
        pltpu.VMEM((NSAMP,), jnp.int32),
        pltpu.SemaphoreType.DMA,
        pltpu.SemaphoreType.DMA,
    ],
    compiler_params=_sc_compiler_params(),
)
def _sample_kernel(xl_hbm, bs_hbm, u_hbm, o_hbm, bs_v, bcum_v, t_v, blk_v,
                   res_v, sem, sem_b):
    _sample_body(xl_hbm, bs_hbm, u_hbm, o_hbm, bs_v, bcum_v, t_v, blk_v,
                 res_v, sem, sem_b)


def kernel(x):
    bs = _pass1(x)
    u = jax.random.uniform(jax.random.key(42), (NROW, NSAMP),
                           dtype=jnp.float32)
    idx = _sample_kernel(x, bs, u)
    return idx.astype(jnp.int64)
